# Initial kernel scaffold; baseline (speedup 1.0000x reference)
import jax
import jax.numpy as jnp
from jax.experimental import pallas as pl


def kernel(x, ref_points, Wv, bv, Woff, boff, Waw, baw, Wo, bo):
    raise NotImplementedError("write your pallas kernel here")


# speedup vs baseline: 18430.3272x; 18430.3272x over previous
import functools
import math

import jax
import jax.numpy as jnp
from jax import lax
from jax.experimental import pallas as pl
from jax.experimental.pallas import tpu as pltpu
from jax.experimental.pallas import tpu_sc as plsc

DIM = 768
P = 8
B = 8
N = 1024
HW = 32

NC, NS = 2, 16
NW = NC * NS
QTOT = B * N
QPW = QTOT // NW
G = 64
NG = QPW // G


def _prep_body(x_ref, refc_ref, wv_ref, bv_ref, wcat_ref, bcat_ref,
               value_ref, w_ref, idx_ref):
    xb = x_ref[0]
    value_ref[0] = jnp.dot(xb, wv_ref[...],
                           preferred_element_type=jnp.float32) + bv_ref[...]
    proj = jnp.dot(xb, wcat_ref[...],
                   preferred_element_type=jnp.float32) + bcat_ref[...]
    g = (jnp.clip(refc_ref[0] + proj[:, 0:16], -1.0, 1.0) + 1.0) * 0.5 * (HW - 1)
    f = jnp.floor(g)
    t = g - f
    fi = f.astype(jnp.int32)
    x0 = fi[:, 0:P]
    y0 = fi[:, P:2 * P]
    tx = t[:, 0:P]
    ty = t[:, P:2 * P]
    x1 = jnp.minimum(x0 + 1, HW - 1)
    y1 = jnp.minimum(y0 + 1, HW - 1)
    logits = proj[:, 16:16 + P]
    m = jnp.max(logits, axis=1, keepdims=True)
    e = jnp.exp(logits - m)
    aw = e / jnp.sum(e, axis=1, keepdims=True)
    w00 = aw * (1.0 - tx) * (1.0 - ty)
    w01 = aw * tx * (1.0 - ty)
    w10 = aw * (1.0 - tx) * ty
    w11 = aw * tx * ty
    i00 = y0 * HW + x0
    i01 = y0 * HW + x1
    i10 = y1 * HW + x0
    i11 = y1 * HW + x1
    w_ref[0] = jnp.concatenate([w00, w01, w10, w11], axis=1)
    idx_ref[0] = jnp.concatenate([i00, i01, i10, i11], axis=1)


def _sc_scatter_body(w_hbm, i_hbm, s_hbm, w_v, i_v, buf):
    wid = lax.axis_index("s") * NC + lax.axis_index("c")
    base = wid * QPW
    K = 4 * P

    def _zero(k, _):
        buf[pl.ds(k * 16, 16)] = jnp.zeros((16,), jnp.float32)
        return 0

    lax.fori_loop(0, G * N // 16, _zero, 0)

    lane = lax.iota(jnp.int32, 16)

    def _group(grp, _):
        qb = base + grp * G
        pltpu.sync_copy(w_hbm.at[pl.ds(qb * K, G * K)], w_v)
        pltpu.sync_copy(i_hbm.at[pl.ds(qb * K, G * K)], i_v)
        for sub in range(G // 16):
            rows = lane + sub * 16
            for pc in range(K):
                addr = rows * K + pc
                wv = plsc.load_gather(w_v, [addr])
                iv = plsc.load_gather(i_v, [addr])
                plsc.addupdate_scatter(buf, [rows * N + iv], wv)
        pltpu.sync_copy(buf, s_hbm.at[pl.ds(qb * N, G * N)])
        for sub in range(G // 16):
            rows = lane + sub * 16
            for pc in range(K):
                iv = plsc.load_gather(i_v, [rows * K + pc])
                plsc.store_scatter(buf, [rows * N + iv],
                                   jnp.zeros((16,), jnp.float32))
        return 0

    lax.fori_loop(0, NG, _group, 0)


def _apply_body(s_ref, value_ref, wo_ref, bo_ref, out_ref):
    attn = jnp.dot(s_ref[0], value_ref[0], preferred_element_type=jnp.float32)
    out_ref[0] = jnp.dot(attn, wo_ref[...],
                         preferred_element_type=jnp.float32) + bo_ref[...]


def kernel(x, ref_points, Wv, bv, Woff, boff, Waw, baw, Wo, bo):
    f32 = jnp.float32
    Wcat = jnp.concatenate([Woff[:, 0::2], Woff[:, 1::2], Waw], axis=1)
    bcat = jnp.concatenate([boff[0::2], boff[1::2], baw])[None, :]
    refc = jnp.concatenate([jnp.tile(ref_points[..., 0:1], (1, 1, P)),
                            jnp.tile(ref_points[..., 1:2], (1, 1, P))], axis=2)

    value, w, idx = pl.pallas_call(
        _prep_body,
        grid=(B,),
        in_specs=[
            pl.BlockSpec((1, N, DIM), lambda b: (b, 0, 0)),
            pl.BlockSpec((1, N, 2 * P), lambda b: (b, 0, 0)),
            pl.BlockSpec((DIM, DIM), lambda b: (0, 0)),
            pl.BlockSpec((1, DIM), lambda b: (0, 0)),
            pl.BlockSpec((DIM, 3 * P), lambda b: (0, 0)),
            pl.BlockSpec((1, 3 * P), lambda b: (0, 0)),
        ],
        out_specs=[
            pl.BlockSpec((1, N, DIM), lambda b: (b, 0, 0)),
            pl.BlockSpec((1, N, 4 * P), lambda b: (b, 0, 0)),
            pl.BlockSpec((1, N, 4 * P), lambda b: (b, 0, 0)),
        ],
        out_shape=[
            jax.ShapeDtypeStruct((B, N, DIM), f32),
            jax.ShapeDtypeStruct((B, N, 4 * P), f32),
            jax.ShapeDtypeStruct((B, N, 4 * P), jnp.int32),
        ],
        compiler_params=pltpu.CompilerParams(
            dimension_semantics=("parallel",)),
    )(x, refc, Wv, bv[None, :], Wcat, bcat)

    wf = w.reshape(QTOT * 4 * P)
    idxf = idx.reshape(QTOT * 4 * P)

    sc_scatter = pl.kernel(
        _sc_scatter_body,
        out_type=jax.ShapeDtypeStruct((QTOT * N,), f32),
        mesh=plsc.VectorSubcoreMesh(core_axis_name="c", subcore_axis_name="s"),
        scratch_types=[
            pltpu.VMEM((G * 4 * P,), f32),
            pltpu.VMEM((G * 4 * P,), jnp.int32),
            pltpu.VMEM((G * N,), f32),
        ],
        compiler_params=pltpu.CompilerParams(needs_layout_passes=False),
    )
    s = sc_scatter(wf, idxf)

    out = pl.pallas_call(
        _apply_body,
        grid=(B,),
        in_specs=[
            pl.BlockSpec((1, N, N), lambda b: (b, 0, 0)),
            pl.BlockSpec((1, N, DIM), lambda b: (b, 0, 0)),
            pl.BlockSpec((DIM, DIM), lambda b: (0, 0)),
            pl.BlockSpec((1, DIM), lambda b: (0, 0)),
        ],
        out_specs=pl.BlockSpec((1, N, DIM), lambda b: (b, 0, 0)),
        out_shape=jax.ShapeDtypeStruct((B, N, DIM), f32),
        compiler_params=pltpu.CompilerParams(
            dimension_semantics=("parallel",)),
    )(s.reshape(B, N, N), value, Wo, bo[None, :])
    return out


# speedup vs baseline: 20717.6989x; 1.1241x over previous
import functools
import math

import jax
import jax.numpy as jnp
from jax import lax
from jax.experimental import pallas as pl
from jax.experimental.pallas import tpu as pltpu
from jax.experimental.pallas import tpu_sc as plsc

DIM = 768
P = 8
B = 8
N = 1024
HW = 32
K = 4 * P

NC, NS = 2, 16
NW = NC * NS
QTOT = B * N
QPW = QTOT // NW
G = 32
NG = QPW // G


def _prep_idx_body(x_ref, refc_ref, wcat_ref, bcat_ref, w_ref, idx_ref):
    proj = jnp.dot(x_ref[0], wcat_ref[...],
                   preferred_element_type=jnp.float32) + bcat_ref[...]
    g = (jnp.clip(refc_ref[0] + proj[:, 0:16], -1.0, 1.0) + 1.0) * 0.5 * (HW - 1)
    f = jnp.floor(g)
    t = g - f
    fi = f.astype(jnp.int32)
    x0 = fi[:, 0:P]
    y0 = fi[:, P:2 * P]
    tx = t[:, 0:P]
    ty = t[:, P:2 * P]
    x1 = jnp.minimum(x0 + 1, HW - 1)
    y1 = jnp.minimum(y0 + 1, HW - 1)
    logits = proj[:, 16:16 + P]
    m = jnp.max(logits, axis=1, keepdims=True)
    e = jnp.exp(logits - m)
    aw = e / jnp.sum(e, axis=1, keepdims=True)
    w00 = aw * (1.0 - tx) * (1.0 - ty)
    w01 = aw * tx * (1.0 - ty)
    w10 = aw * (1.0 - tx) * ty
    w11 = aw * tx * ty
    i00 = y0 * HW + x0
    i01 = y0 * HW + x1
    i10 = y1 * HW + x0
    i11 = y1 * HW + x1
    w_ref[0] = jnp.concatenate([w00, w01, w10, w11], axis=1)
    idx_ref[0] = jnp.concatenate([i00, i01, i10, i11], axis=1)


def _prep_value_body(x_ref, wv_ref, bv_ref, value_ref):
    value_ref[0] = jnp.dot(x_ref[0], wv_ref[...],
                           preferred_element_type=jnp.float32) + bv_ref[...]


def _sc_scatter_body(w_hbm, i_hbm, s_hbm, w_v, i_v0, i_v1, buf0, buf1,
                     sem0, sem1):
    wid = lax.axis_index("s") * NC + lax.axis_index("c")
    base = wid * QPW
    bufs = (buf0, buf1)
    ivs = (i_v0, i_v1)
    sems = (sem0, sem1)
    zero16 = jnp.zeros((16,), jnp.float32)

    def _zero(k, _):
        for u in range(16):
            off = k * 256 + u * 16
            buf0[pl.ds(off, 16)] = zero16
            buf1[pl.ds(off, 16)] = zero16
        return 0

    lax.fori_loop(0, G * N // 256, _zero, 0)

    lane = lax.iota(jnp.int32, 16)
    copies = [None, None]
    for grp in range(NG):
        slot = grp % 2
        buf = bufs[slot]
        i_v = ivs[slot]
        qb = base + grp * G
        if copies[slot] is not None:
            copies[slot].wait()
            for sub in range(G // 16):
                rows = lane + sub * 16
                for pc in range(K):
                    iv = plsc.load_gather(i_v, [rows * K + pc])
                    plsc.store_scatter(buf, [rows * N + iv], zero16)
        pltpu.sync_copy(w_hbm.at[pl.ds(qb * K, G * K)], w_v)
        pltpu.sync_copy(i_hbm.at[pl.ds(qb * K, G * K)], i_v)
        for sub in range(G // 16):
            rows = lane + sub * 16
            for pc in range(K):
                addr = rows * K + pc
                wv = plsc.load_gather(w_v, [addr])
                iv = plsc.load_gather(i_v, [addr])
                plsc.addupdate_scatter(buf, [rows * N + iv], wv)
        copies[slot] = pltpu.async_copy(buf, s_hbm.at[pl.ds(qb * N, G * N)],
                                        sems[slot])
    copies[0].wait()
    copies[1].wait()


def _apply_body(s_ref, value_ref, wo_ref, bo_ref, out_ref):
    attn = jnp.dot(s_ref[0], value_ref[0], preferred_element_type=jnp.float32)
    out_ref[0] = jnp.dot(attn, wo_ref[...],
                         preferred_element_type=jnp.float32) + bo_ref[...]


def kernel(x, ref_points, Wv, bv, Woff, boff, Waw, baw, Wo, bo):
    f32 = jnp.float32
    Wcat = jnp.concatenate([Woff[:, 0::2], Woff[:, 1::2], Waw], axis=1)
    bcat = jnp.concatenate([boff[0::2], boff[1::2], baw])[None, :]
    refc = jnp.concatenate([jnp.tile(ref_points[..., 0:1], (1, 1, P)),
                            jnp.tile(ref_points[..., 1:2], (1, 1, P))], axis=2)

    w, idx = pl.pallas_call(
        _prep_idx_body,
        grid=(B,),
        in_specs=[
            pl.BlockSpec((1, N, DIM), lambda b: (b, 0, 0)),
            pl.BlockSpec((1, N, 2 * P), lambda b: (b, 0, 0)),
            pl.BlockSpec((DIM, 3 * P), lambda b: (0, 0)),
            pl.BlockSpec((1, 3 * P), lambda b: (0, 0)),
        ],
        out_specs=[
            pl.BlockSpec((1, N, K), lambda b: (b, 0, 0)),
            pl.BlockSpec((1, N, K), lambda b: (b, 0, 0)),
        ],
        out_shape=[
            jax.ShapeDtypeStruct((B, N, K), f32),
            jax.ShapeDtypeStruct((B, N, K), jnp.int32),
        ],
        compiler_params=pltpu.CompilerParams(
            dimension_semantics=("parallel",)),
    )(x, refc, Wcat, bcat)

    sc_scatter = pl.kernel(
        _sc_scatter_body,
        out_type=jax.ShapeDtypeStruct((QTOT * N,), f32),
        mesh=plsc.VectorSubcoreMesh(core_axis_name="c", subcore_axis_name="s"),
        scratch_types=[
            pltpu.VMEM((G * K,), f32),
            pltpu.VMEM((G * K,), jnp.int32),
            pltpu.VMEM((G * K,), jnp.int32),
            pltpu.VMEM((G * N,), f32),
            pltpu.VMEM((G * N,), f32),
            pltpu.SemaphoreType.DMA,
            pltpu.SemaphoreType.DMA,
        ],
        compiler_params=pltpu.CompilerParams(needs_layout_passes=False),
    )
    s = sc_scatter(w.reshape(QTOT * K), idx.reshape(QTOT * K))

    value = pl.pallas_call(
        _prep_value_body,
        grid=(B,),
        in_specs=[
            pl.BlockSpec((1, N, DIM), lambda b: (b, 0, 0)),
            pl.BlockSpec((DIM, DIM), lambda b: (0, 0)),
            pl.BlockSpec((1, DIM), lambda b: (0, 0)),
        ],
        out_specs=pl.BlockSpec((1, N, DIM), lambda b: (b, 0, 0)),
        out_shape=jax.ShapeDtypeStruct((B, N, DIM), f32),
        compiler_params=pltpu.CompilerParams(
            dimension_semantics=("parallel",)),
    )(x, Wv, bv[None, :])

    out = pl.pallas_call(
        _apply_body,
        grid=(B,),
        in_specs=[
            pl.BlockSpec((1, N, N), lambda b: (b, 0, 0)),
            pl.BlockSpec((1, N, DIM), lambda b: (b, 0, 0)),
            pl.BlockSpec((DIM, DIM), lambda b: (0, 0)),
            pl.BlockSpec((1, DIM), lambda b: (0, 0)),
        ],
        out_specs=pl.BlockSpec((1, N, DIM), lambda b: (b, 0, 0)),
        out_shape=jax.ShapeDtypeStruct((B, N, DIM), f32),
        compiler_params=pltpu.CompilerParams(
            dimension_semantics=("parallel",)),
    )(s.reshape(B, N, N), value, Wo, bo[None, :])
    return out


# speedup vs baseline: 23499.1254x; 1.1343x over previous
import functools
import math

import jax
import jax.numpy as jnp
from jax import lax
from jax.experimental import pallas as pl
from jax.experimental.pallas import tpu as pltpu
from jax.experimental.pallas import tpu_sc as plsc

DIM = 768
P = 8
B = 8
N = 1024
HW = 32
K = 4 * P

NC, NS = 2, 16
NW = NC * NS
QTOT = B * N
QPW = QTOT // NW
G = 32
NG = QPW // G


def _prep_idx_body(x_ref, refc_ref, wcat_ref, bcat_ref, w_ref, idx_ref):
    proj = jnp.dot(x_ref[0], wcat_ref[...],
                   preferred_element_type=jnp.float32) + bcat_ref[...]
    g = (jnp.clip(refc_ref[0] + proj[:, 0:16], -1.0, 1.0) + 1.0) * 0.5 * (HW - 1)
    f = jnp.floor(g)
    t = g - f
    fi = f.astype(jnp.int32)
    x0 = fi[:, 0:P]
    y0 = fi[:, P:2 * P]
    tx = t[:, 0:P]
    ty = t[:, P:2 * P]
    x1 = jnp.minimum(x0 + 1, HW - 1)
    y1 = jnp.minimum(y0 + 1, HW - 1)
    logits = proj[:, 16:16 + P]
    m = jnp.max(logits, axis=1, keepdims=True)
    e = jnp.exp(logits - m)
    aw = e / jnp.sum(e, axis=1, keepdims=True)
    w00 = aw * (1.0 - tx) * (1.0 - ty)
    w01 = aw * tx * (1.0 - ty)
    w10 = aw * (1.0 - tx) * ty
    w11 = aw * tx * ty
    i00 = y0 * HW + x0
    i01 = y0 * HW + x1
    i10 = y1 * HW + x0
    i11 = y1 * HW + x1
    w_ref[0] = jnp.concatenate([w00, w01, w10, w11], axis=1)
    idx_ref[0] = jnp.concatenate([i00, i01, i10, i11], axis=1)


def _sc_scatter_body(w_hbm, i_hbm, s_hbm, w_v, i_v0, i_v1, buf0, buf1,
                     sem0, sem1):
    wid = lax.axis_index("s") * NC + lax.axis_index("c")
    base = wid * QPW
    bufs = (buf0, buf1)
    ivs = (i_v0, i_v1)
    sems = (sem0, sem1)
    zero16 = jnp.zeros((16,), jnp.float32)

    def _zero(k, _):
        for u in range(16):
            off = k * 256 + u * 16
            buf0[pl.ds(off, 16)] = zero16
            buf1[pl.ds(off, 16)] = zero16
        return 0

    lax.fori_loop(0, G * N // 256, _zero, 0)

    lane = lax.iota(jnp.int32, 16)
    copies = [None, None]
    for grp in range(NG):
        slot = grp % 2
        buf = bufs[slot]
        i_v = ivs[slot]
        qb = base + grp * G
        if copies[slot] is not None:
            copies[slot].wait()
            for sub in range(G // 16):
                rows = lane + sub * 16
                for pc in range(K):
                    iv = plsc.load_gather(i_v, [rows * K + pc])
                    plsc.store_scatter(buf, [rows * N + iv], zero16)
        pltpu.sync_copy(w_hbm.at[pl.ds(qb * K, G * K)], w_v)
        pltpu.sync_copy(i_hbm.at[pl.ds(qb * K, G * K)], i_v)
        for sub in range(G // 16):
            rows = lane + sub * 16
            for pc in range(K):
                addr = rows * K + pc
                wv = plsc.load_gather(w_v, [addr])
                iv = plsc.load_gather(i_v, [addr])
                plsc.addupdate_scatter(buf, [rows * N + iv], wv)
        copies[slot] = pltpu.async_copy(buf, s_hbm.at[pl.ds(qb * N, G * N)],
                                        sems[slot])
    copies[0].wait()
    copies[1].wait()


def _apply_body(s_ref, x_ref, wv_ref, bv_ref, wo_ref, bo_ref, out_ref):
    value = jnp.dot(x_ref[0], wv_ref[...],
                    preferred_element_type=jnp.float32) + bv_ref[...]
    attn = jnp.dot(s_ref[0], value, preferred_element_type=jnp.float32)
    out_ref[0] = jnp.dot(attn, wo_ref[...],
                         preferred_element_type=jnp.float32) + bo_ref[...]


def kernel(x, ref_points, Wv, bv, Woff, boff, Waw, baw, Wo, bo):
    f32 = jnp.float32
    Wcat = jnp.concatenate([Woff[:, 0::2], Woff[:, 1::2], Waw], axis=1)
    bcat = jnp.concatenate([boff[0::2], boff[1::2], baw])[None, :]
    refc = jnp.concatenate([jnp.tile(ref_points[..., 0:1], (1, 1, P)),
                            jnp.tile(ref_points[..., 1:2], (1, 1, P))], axis=2)

    w, idx = pl.pallas_call(
        _prep_idx_body,
        grid=(B,),
        in_specs=[
            pl.BlockSpec((1, N, DIM), lambda b: (b, 0, 0)),
            pl.BlockSpec((1, N, 2 * P), lambda b: (b, 0, 0)),
            pl.BlockSpec((DIM, 3 * P), lambda b: (0, 0)),
            pl.BlockSpec((1, 3 * P), lambda b: (0, 0)),
        ],
        out_specs=[
            pl.BlockSpec((1, N, K), lambda b: (b, 0, 0)),
            pl.BlockSpec((1, N, K), lambda b: (b, 0, 0)),
        ],
        out_shape=[
            jax.ShapeDtypeStruct((B, N, K), f32),
            jax.ShapeDtypeStruct((B, N, K), jnp.int32),
        ],
        compiler_params=pltpu.CompilerParams(
            dimension_semantics=("parallel",)),
    )(x, refc, Wcat, bcat)

    sc_scatter = pl.kernel(
        _sc_scatter_body,
        out_type=jax.ShapeDtypeStruct((QTOT * N,), f32),
        mesh=plsc.VectorSubcoreMesh(core_axis_name="c", subcore_axis_name="s"),
        scratch_types=[
            pltpu.VMEM((G * K,), f32),
            pltpu.VMEM((G * K,), jnp.int32),
            pltpu.VMEM((G * K,), jnp.int32),
            pltpu.VMEM((G * N,), f32),
            pltpu.VMEM((G * N,), f32),
            pltpu.SemaphoreType.DMA,
            pltpu.SemaphoreType.DMA,
        ],
        compiler_params=pltpu.CompilerParams(needs_layout_passes=False),
    )
    s = sc_scatter(w.reshape(QTOT * K), idx.reshape(QTOT * K))

    out = pl.pallas_call(
        _apply_body,
        grid=(B,),
        in_specs=[
            pl.BlockSpec((1, N, N), lambda b: (b, 0, 0)),
            pl.BlockSpec((1, N, DIM), lambda b: (b, 0, 0)),
            pl.BlockSpec((DIM, DIM), lambda b: (0, 0)),
            pl.BlockSpec((1, DIM), lambda b: (0, 0)),
            pl.BlockSpec((DIM, DIM), lambda b: (0, 0)),
            pl.BlockSpec((1, DIM), lambda b: (0, 0)),
        ],
        out_specs=pl.BlockSpec((1, N, DIM), lambda b: (b, 0, 0)),
        out_shape=jax.ShapeDtypeStruct((B, N, DIM), f32),
        compiler_params=pltpu.CompilerParams(
            dimension_semantics=("parallel",)),
    )(s.reshape(B, N, N), x, Wv, bv[None, :], Wo, bo[None, :])
    return out
